# baseline (speedup 1.0000x reference)
import functools

import jax
import jax.numpy as jnp
from jax.experimental import pallas as pl
from jax.experimental.pallas import tpu as pltpu

CONFIDENCE = 0.005
BATCH = 16384
NCLS = 1000
BLOCK_R = 512


def _loss_kernel(a_ref, b_ref, out_ref, acc_ref):
    i = pl.program_id(0)
    nsteps = pl.num_programs(0)

    a = a_ref[...]
    b = b_ref[...]

    m1 = jnp.max(a, axis=1, keepdims=True)
    e1 = jnp.exp(a - m1)
    sum1 = jnp.sum(e1, axis=1, keepdims=True)
    mask = (1.0 / sum1) > CONFIDENCE

    cols = jax.lax.broadcasted_iota(jnp.int32, a.shape, 1)
    target = jnp.min(jnp.where(a == m1, cols, NCLS), axis=1, keepdims=True)

    m2 = jnp.max(b, axis=1, keepdims=True)
    sum2 = jnp.sum(jnp.exp(b - m2), axis=1, keepdims=True)
    lse2 = m2 + jnp.log(sum2)
    eqf = jnp.where(cols == target, 1.0, 0.0)
    val = jnp.sum(eqf * b, axis=1, keepdims=True)
    nll = lse2 - val

    maskf = jnp.where(mask, 1.0, 0.0)
    g = jnp.concatenate([maskf, maskf * nll], axis=1)
    part = jax.lax.dot_general(
        g, eqf, (((0,), (0,)), ((), ())),
        preferred_element_type=jnp.float32,
    )

    @pl.when(i == 0)
    def _init():
        acc_ref[...] = part

    @pl.when(i > 0)
    def _acc():
        acc_ref[...] += part

    @pl.when(i == nsteps - 1)
    def _finish():
        counts = acc_ref[0:1, :]
        s = acc_ref[1:2, :]
        used = counts > 0.0
        weight = jnp.where(used, BATCH / jnp.maximum(counts, 1.0), 0.0)
        numer = jnp.sum(weight * s, keepdims=True)
        denom = jnp.sum(weight * counts, keepdims=True)
        out_ref[...] = (numer / denom).reshape(1, 1)


@jax.jit
def kernel(anchors, anchors_aug):
    grid = BATCH // BLOCK_R
    out = pl.pallas_call(
        _loss_kernel,
        grid=(grid,),
        in_specs=[
            pl.BlockSpec((BLOCK_R, NCLS), lambda i: (i, 0)),
            pl.BlockSpec((BLOCK_R, NCLS), lambda i: (i, 0)),
        ],
        out_specs=pl.BlockSpec((1, 1), lambda i: (0, 0)),
        out_shape=jax.ShapeDtypeStruct((1, 1), jnp.float32),
        scratch_shapes=[
            pltpu.VMEM((2, NCLS), jnp.float32),
        ],
    )(anchors, anchors_aug)
    return out[0, 0]


# speedup vs baseline: 1.0467x; 1.0467x over previous
import functools

import jax
import jax.numpy as jnp
from jax.experimental import pallas as pl
from jax.experimental.pallas import tpu as pltpu

CONFIDENCE = 0.005
BATCH = 16384
NCLS = 1000
BLOCK_R = 1024


def _loss_kernel(a_ref, b_ref, out_ref, acc_ref):
    i = pl.program_id(0)
    nsteps = pl.num_programs(0)

    a = a_ref[...]
    b = b_ref[...]

    m1 = jnp.max(a, axis=1, keepdims=True)
    e1 = jnp.exp(a - m1)
    sum1 = jnp.sum(e1, axis=1, keepdims=True)
    mask = (1.0 / sum1) > CONFIDENCE

    cols = jax.lax.broadcasted_iota(jnp.int32, a.shape, 1)
    target = jnp.min(jnp.where(a == m1, cols, NCLS), axis=1, keepdims=True)

    m2 = jnp.max(b, axis=1, keepdims=True)
    sum2 = jnp.sum(jnp.exp(b - m2), axis=1, keepdims=True)
    lse2 = m2 + jnp.log(sum2)
    eqf = jnp.where(cols == target, 1.0, 0.0)
    val = jnp.sum(eqf * b, axis=1, keepdims=True)
    nll = lse2 - val

    maskf = jnp.where(mask, 1.0, 0.0)
    g = jnp.concatenate([maskf, maskf * nll], axis=1)
    part = jax.lax.dot_general(
        g, eqf, (((0,), (0,)), ((), ())),
        preferred_element_type=jnp.float32,
    )

    @pl.when(i == 0)
    def _init():
        acc_ref[...] = part

    @pl.when(i > 0)
    def _acc():
        acc_ref[...] += part

    @pl.when(i == nsteps - 1)
    def _finish():
        counts = acc_ref[0:1, :]
        s = acc_ref[1:2, :]
        used = counts > 0.0
        weight = jnp.where(used, BATCH / jnp.maximum(counts, 1.0), 0.0)
        numer = jnp.sum(weight * s, keepdims=True)
        denom = jnp.sum(weight * counts, keepdims=True)
        out_ref[...] = (numer / denom).reshape(1, 1)


@jax.jit
def kernel(anchors, anchors_aug):
    grid = BATCH // BLOCK_R
    out = pl.pallas_call(
        _loss_kernel,
        grid=(grid,),
        in_specs=[
            pl.BlockSpec((BLOCK_R, NCLS), lambda i: (i, 0)),
            pl.BlockSpec((BLOCK_R, NCLS), lambda i: (i, 0)),
        ],
        out_specs=pl.BlockSpec((1, 1), lambda i: (0, 0)),
        out_shape=jax.ShapeDtypeStruct((1, 1), jnp.float32),
        scratch_shapes=[
            pltpu.VMEM((2, NCLS), jnp.float32),
        ],
    )(anchors, anchors_aug)
    return out[0, 0]
